# baseline (speedup 1.0000x reference)
import functools

import jax
import jax.numpy as jnp
from jax import lax
from jax.experimental import pallas as pl
from jax.experimental.pallas import tpu as pltpu
from jax.experimental.pallas import tpu_sc as plsc

N = 10000
D = 128
H = 128
C = 10
E = 320000

NC = 2
NS = 16
L = 16
DH = D // NC
DW = DH + 1

K = 128
E_PAD = 327680
CH = E_PAD // (NS * K)
N_ACC = 10240
RPT = N_ACC // NS


def _sc_aggregate_body(table_h, src_h, dst_h, acc_out,
                       src_v, dst_v, rows0, rows1, zbuf, acc_sh, sem0, sem1):
  c = lax.axis_index("c")
  s = lax.axis_index("s")
  base = s * RPT
  zero16 = jnp.zeros((L,), jnp.float32)

  def zrow(r, _):
    for jj in (0, 16, 32, 48, DW - L):
      zbuf[r, pl.ds(jj, L)] = zero16
    return 0
  lax.fori_loop(0, 128, zrow, 0)
  for kk in range(RPT // 128):
    pltpu.sync_copy(zbuf, acc_sh.at[pl.ds(base + kk * 128, 128)])

  pltpu.sync_copy(src_h.at[c, s], src_v)
  pltpu.sync_copy(dst_h.at[s], dst_v)
  plsc.subcore_barrier()

  def issue(j, buf, sem):
    pltpu.async_copy(table_h.at[src_v.at[j]], buf, sem)

  def wait(buf, sem):
    pltpu.make_async_copy(table_h.at[src_v.at[0]], buf, sem).wait()

  def scatter(j, buf):
    pltpu.sync_copy(buf, acc_sh.at[dst_v.at[j]], add=True)

  issue(0, rows0, sem0)
  issue(1, rows1, sem1)

  def pair(i, _):
    j = i * 2
    wait(rows0, sem0)
    scatter(j, rows0)
    issue(j + 2, rows0, sem0)
    wait(rows1, sem1)
    scatter(j + 1, rows1)
    issue(j + 3, rows1, sem1)
    return 0
  lax.fori_loop(0, (CH - 2) // 2, pair, 0)
  wait(rows0, sem0)
  scatter(CH - 2, rows0)
  wait(rows1, sem1)
  scatter(CH - 1, rows1)

  plsc.subcore_barrier()
  pltpu.sync_copy(acc_sh.at[pl.ds(base, RPT)], acc_out.at[c, pl.ds(base, RPT)])


_sc_aggregate = pl.kernel(
    _sc_aggregate_body,
    out_type=[jax.ShapeDtypeStruct((NC, N_ACC, DW), jnp.float32)],
    mesh=plsc.VectorSubcoreMesh(
        core_axis_name="c", subcore_axis_name="s", num_cores=NC, num_subcores=NS
    ),
    scratch_types=[
        pltpu.VMEM((CH, K), jnp.int32),
        pltpu.VMEM((CH, K), jnp.int32),
        pltpu.VMEM((K, DW), jnp.float32),
        pltpu.VMEM((K, DW), jnp.float32),
        pltpu.VMEM((128, DW), jnp.float32),
        pltpu.VMEM_SHARED((N_ACC, DW), jnp.float32),
        pltpu.SemaphoreType.DMA,
        pltpu.SemaphoreType.DMA,
    ],
    compiler_params=pltpu.CompilerParams(use_tc_tiling_on_sc=False),
)


BR = 2000
NG = N // BR


def _tc_layer1_body(x_b, acc_b, ws_b, wn_b, b_b, out_b):
  xv = x_b[...]
  accv = acc_b[...]
  x = jnp.concatenate([xv[0, :, :DH], xv[1, :, :DH]], axis=1)
  acc = jnp.concatenate([accv[0, :, :DH], accv[1, :, :DH]], axis=1)
  deg = jnp.maximum(accv[0, :, DH:DW], 1.0)
  mean = acc / deg
  h = x @ ws_b[...] + mean @ wn_b[...] + b_b[...]
  h = jnp.maximum(h, 0.0)
  one = jnp.ones((h.shape[0], 1), jnp.float32)
  out_b[...] = jnp.stack(
      [jnp.concatenate([h[:, :DH], one], axis=1),
       jnp.concatenate([h[:, DH:], one], axis=1)])


def _tc_layer1(xs, acc, w_self, w_neigh, b):
  return pl.pallas_call(
      _tc_layer1_body,
      grid=(NG,),
      in_specs=[
          pl.BlockSpec((NC, BR, DW), lambda i: (0, i, 0)),
          pl.BlockSpec((NC, BR, DW), lambda i: (0, i, 0)),
          pl.BlockSpec((D, H), lambda i: (0, 0)),
          pl.BlockSpec((D, H), lambda i: (0, 0)),
          pl.BlockSpec((1, H), lambda i: (0, 0)),
      ],
      out_specs=pl.BlockSpec((NC, BR, DW), lambda i: (0, i, 0)),
      out_shape=jax.ShapeDtypeStruct((NC, N, DW), jnp.float32),
  )(xs, acc, w_self, w_neigh, b)


def _tc_layer2_body(h_b, acc_b, ws_b, wn_b, b_b, wc_b, bc_b, out_b, sum_ref):
  i = pl.program_id(0)
  hv = h_b[...]
  accv = acc_b[...]
  h1 = jnp.concatenate([hv[0, :, :DH], hv[1, :, :DH]], axis=1)
  acc = jnp.concatenate([accv[0, :, :DH], accv[1, :, :DH]], axis=1)
  deg = jnp.maximum(accv[0, :, DH:DW], 1.0)
  mean = acc / deg
  h = h1 @ ws_b[...] + mean @ wn_b[...] + b_b[...]
  h = jnp.maximum(h, 0.0)

  @pl.when(i == 0)
  def _():
    sum_ref[...] = jnp.zeros_like(sum_ref)

  sum_ref[...] += jnp.sum(h, axis=0, keepdims=True)

  @pl.when(i == NG - 1)
  def _():
    hg = sum_ref[...] * (1.0 / N)
    out_b[...] = hg @ wc_b[...] + bc_b[...]


def _tc_layer2(h1s, acc, w_self, w_neigh, b, w_cls, b_cls):
  return pl.pallas_call(
      _tc_layer2_body,
      grid=(NG,),
      in_specs=[
          pl.BlockSpec((NC, BR, DW), lambda i: (0, i, 0)),
          pl.BlockSpec((NC, BR, DW), lambda i: (0, i, 0)),
          pl.BlockSpec((H, H), lambda i: (0, 0)),
          pl.BlockSpec((H, H), lambda i: (0, 0)),
          pl.BlockSpec((1, H), lambda i: (0, 0)),
          pl.BlockSpec((H, C), lambda i: (0, 0)),
          pl.BlockSpec((1, C), lambda i: (0, 0)),
      ],
      out_specs=pl.BlockSpec((1, C), lambda i: (0, 0)),
      out_shape=jax.ShapeDtypeStruct((1, C), jnp.float32),
      scratch_shapes=[pltpu.VMEM((1, H), jnp.float32)],
  )(h1s, acc, w_self, w_neigh, b, w_cls, b_cls)


@jax.jit
def kernel(x, edge_index, W_self1, W_neigh1, b1, W_self2, W_neigh2, b2, W_cls, b_cls):
  src = edge_index[0]
  dst = edge_index[1]
  pad = E_PAD - E
  src_p = jnp.concatenate([src, jnp.zeros((pad,), jnp.int32)])
  dst_p = jnp.concatenate([dst, jnp.full((pad,), N, jnp.int32)])
  src4 = (src_p[None, :] + (jnp.arange(NC, dtype=jnp.int32) * N)[:, None])
  src4 = src4.reshape(NC, NS, CH, K)
  dst3 = dst_p.reshape(NS, CH, K)

  one = jnp.ones((N, 1), jnp.float32)
  xs = jnp.stack([jnp.concatenate([x[:, :DH], one], axis=1),
                  jnp.concatenate([x[:, DH:], one], axis=1)])
  (acc1,) = _sc_aggregate(xs.reshape(NC * N, DW), src4, dst3)
  h1s = _tc_layer1(xs, acc1, W_self1, W_neigh1, b1.reshape(1, H))
  (acc2,) = _sc_aggregate(h1s.reshape(NC * N, DW), src4, dst3)
  return _tc_layer2(h1s, acc2, W_self2, W_neigh2, b2.reshape(1, H),
                    W_cls, b_cls.reshape(1, C))


# speedup vs baseline: 1.2843x; 1.2843x over previous
import functools

import jax
import jax.numpy as jnp
from jax import lax
from jax.experimental import pallas as pl
from jax.experimental.pallas import tpu as pltpu
from jax.experimental.pallas import tpu_sc as plsc

N = 10000
D = 128
H = 128
C = 10
E = 320000

NC = 2
NS = 16
L = 16
DH = D // NC

K = 128
E_PAD = 327680
CH = E_PAD // (NS * K)
N_ACC = 10240
RPT = N_ACC // NS


def _sc_aggregate_body(table_h, src_h, dst_h, acc_out, deg_out,
                       src_v, dst_v, rows0, rows1, zbuf, ones_v, zdeg,
                       acc_sh, deg_sh, sem0, sem1, osem):
  c = lax.axis_index("c")
  s = lax.axis_index("s")
  base = s * RPT
  zero16 = jnp.zeros((L,), jnp.float32)

  def zrow(r, _):
    for jj in range(DH // L):
      zbuf[r, pl.ds(jj * L, L)] = zero16
    ones_v[r, :] = jnp.ones((L,), jnp.float32)
    return 0
  lax.fori_loop(0, 128, zrow, 0)

  def zdrow(r, _):
    zdeg[r, :] = zero16
    return 0
  lax.fori_loop(0, RPT, zdrow, 0)

  pltpu.sync_copy(zdeg, deg_sh.at[pl.ds(base, RPT)])
  for kk in range(RPT // 128):
    pltpu.sync_copy(zbuf, acc_sh.at[pl.ds(base + kk * 128, 128)])

  pltpu.sync_copy(src_h.at[c, s], src_v)
  pltpu.sync_copy(dst_h.at[s], dst_v)
  plsc.subcore_barrier()

  def issue(j, buf, sem):
    pltpu.async_copy(table_h.at[src_v.at[j]], buf, sem)

  def wait(buf, sem):
    pltpu.make_async_copy(table_h.at[src_v.at[0]], buf, sem).wait()

  def scatter(j, buf):
    pltpu.async_copy(ones_v, deg_sh.at[dst_v.at[j]], osem, add=True)
    pltpu.sync_copy(buf, acc_sh.at[dst_v.at[j]], add=True)

  issue(0, rows0, sem0)
  issue(1, rows1, sem1)

  def pair(i, _):
    j = i * 2
    wait(rows0, sem0)
    scatter(j, rows0)
    issue(j + 2, rows0, sem0)
    wait(rows1, sem1)
    scatter(j + 1, rows1)
    issue(j + 3, rows1, sem1)
    return 0
  lax.fori_loop(0, (CH - 2) // 2, pair, 0)
  wait(rows0, sem0)
  scatter(CH - 2, rows0)
  wait(rows1, sem1)
  scatter(CH - 1, rows1)

  def drain(i, _):
    pltpu.make_async_copy(ones_v, deg_sh.at[dst_v.at[0]], osem).wait()
    return 0
  lax.fori_loop(0, CH, drain, 0)

  plsc.subcore_barrier()
  pltpu.sync_copy(acc_sh.at[pl.ds(base, RPT)], acc_out.at[c, pl.ds(base, RPT)])
  pltpu.sync_copy(deg_sh.at[pl.ds(base, RPT)], deg_out.at[c, pl.ds(base, RPT)])


_sc_aggregate = pl.kernel(
    _sc_aggregate_body,
    out_type=[
        jax.ShapeDtypeStruct((NC, N_ACC, DH), jnp.float32),
        jax.ShapeDtypeStruct((NC, N_ACC, L), jnp.float32),
    ],
    mesh=plsc.VectorSubcoreMesh(
        core_axis_name="c", subcore_axis_name="s", num_cores=NC, num_subcores=NS
    ),
    scratch_types=[
        pltpu.VMEM((CH, K), jnp.int32),
        pltpu.VMEM((CH, K), jnp.int32),
        pltpu.VMEM((K, DH), jnp.float32),
        pltpu.VMEM((K, DH), jnp.float32),
        pltpu.VMEM((128, DH), jnp.float32),
        pltpu.VMEM((K, L), jnp.float32),
        pltpu.VMEM((RPT, L), jnp.float32),
        pltpu.VMEM_SHARED((N_ACC, DH), jnp.float32),
        pltpu.VMEM_SHARED((N_ACC, L), jnp.float32),
        pltpu.SemaphoreType.DMA,
        pltpu.SemaphoreType.DMA,
        pltpu.SemaphoreType.DMA,
    ],
    compiler_params=pltpu.CompilerParams(use_tc_tiling_on_sc=False),
)


BR = 2000
NG = N // BR


def _tc_layer1_body(x_b, acc_b, deg_b, ws_b, wn_b, b_b, out_b):
  xv = x_b[...]
  accv = acc_b[...]
  x = jnp.concatenate([xv[0], xv[1]], axis=1)
  acc = jnp.concatenate([accv[0], accv[1]], axis=1)
  deg = jnp.maximum(deg_b[0, :, 0:1], 1.0)
  mean = acc / deg
  h = x @ ws_b[...] + mean @ wn_b[...] + b_b[...]
  h = jnp.maximum(h, 0.0)
  out_b[...] = jnp.stack([h[:, :DH], h[:, DH:]])


def _tc_layer1(xs, acc, deg, w_self, w_neigh, b):
  return pl.pallas_call(
      _tc_layer1_body,
      grid=(NG,),
      in_specs=[
          pl.BlockSpec((NC, BR, DH), lambda i: (0, i, 0)),
          pl.BlockSpec((NC, BR, DH), lambda i: (0, i, 0)),
          pl.BlockSpec((1, BR, L), lambda i: (0, i, 0)),
          pl.BlockSpec((D, H), lambda i: (0, 0)),
          pl.BlockSpec((D, H), lambda i: (0, 0)),
          pl.BlockSpec((1, H), lambda i: (0, 0)),
      ],
      out_specs=pl.BlockSpec((NC, BR, DH), lambda i: (0, i, 0)),
      out_shape=jax.ShapeDtypeStruct((NC, N, DH), jnp.float32),
  )(xs, acc, deg, w_self, w_neigh, b)


def _tc_layer2_body(h_b, acc_b, deg_b, ws_b, wn_b, b_b, wc_b, bc_b, out_b, sum_ref):
  i = pl.program_id(0)
  hv = h_b[...]
  accv = acc_b[...]
  h1 = jnp.concatenate([hv[0], hv[1]], axis=1)
  acc = jnp.concatenate([accv[0], accv[1]], axis=1)
  deg = jnp.maximum(deg_b[0, :, 0:1], 1.0)
  mean = acc / deg
  h = h1 @ ws_b[...] + mean @ wn_b[...] + b_b[...]
  h = jnp.maximum(h, 0.0)

  @pl.when(i == 0)
  def _():
    sum_ref[...] = jnp.zeros_like(sum_ref)

  sum_ref[...] += jnp.sum(h, axis=0, keepdims=True)

  @pl.when(i == NG - 1)
  def _():
    hg = sum_ref[...] * (1.0 / N)
    out_b[...] = hg @ wc_b[...] + bc_b[...]


def _tc_layer2(h1s, acc, deg, w_self, w_neigh, b, w_cls, b_cls):
  return pl.pallas_call(
      _tc_layer2_body,
      grid=(NG,),
      in_specs=[
          pl.BlockSpec((NC, BR, DH), lambda i: (0, i, 0)),
          pl.BlockSpec((NC, BR, DH), lambda i: (0, i, 0)),
          pl.BlockSpec((1, BR, L), lambda i: (0, i, 0)),
          pl.BlockSpec((H, H), lambda i: (0, 0)),
          pl.BlockSpec((H, H), lambda i: (0, 0)),
          pl.BlockSpec((1, H), lambda i: (0, 0)),
          pl.BlockSpec((H, C), lambda i: (0, 0)),
          pl.BlockSpec((1, C), lambda i: (0, 0)),
      ],
      out_specs=pl.BlockSpec((1, C), lambda i: (0, 0)),
      out_shape=jax.ShapeDtypeStruct((1, C), jnp.float32),
      scratch_shapes=[pltpu.VMEM((1, H), jnp.float32)],
  )(h1s, acc, deg, w_self, w_neigh, b, w_cls, b_cls)


@jax.jit
def kernel(x, edge_index, W_self1, W_neigh1, b1, W_self2, W_neigh2, b2, W_cls, b_cls):
  src = edge_index[0]
  dst = edge_index[1]
  pad = E_PAD - E
  src_p = jnp.concatenate([src, jnp.zeros((pad,), jnp.int32)])
  dst_p = jnp.concatenate([dst, jnp.full((pad,), N, jnp.int32)])
  src4 = (src_p[None, :] + (jnp.arange(NC, dtype=jnp.int32) * N)[:, None])
  src4 = src4.reshape(NC, NS, CH, K)
  dst3 = dst_p.reshape(NS, CH, K)

  xs = jnp.stack([x[:, :DH], x[:, DH:]])
  acc1, deg = _sc_aggregate(xs.reshape(NC * N, DH), src4, dst3)
  h1s = _tc_layer1(xs, acc1, deg, W_self1, W_neigh1, b1.reshape(1, H))
  acc2, _ = _sc_aggregate(h1s.reshape(NC * N, DH), src4, dst3)
  return _tc_layer2(h1s, acc2, deg, W_self2, W_neigh2, b2.reshape(1, H),
                    W_cls, b_cls.reshape(1, C))


# speedup vs baseline: 1.3499x; 1.0511x over previous
import functools

import numpy as np
import jax
import jax.numpy as jnp
from jax import lax
from jax.experimental import pallas as pl
from jax.experimental.pallas import tpu as pltpu
from jax.experimental.pallas import tpu_sc as plsc

N = 10000
D = 128
H = 128
C = 10
E = 320000

NC = 2
NS = 16
L = 16
DH = D // NC

K = 128
E_PAD = 327680
CH = E_PAD // (NS * K)
N_ACC = 10240
RPT = N_ACC // NS

_T64 = [(a // 32) * 32 + (2 * (a % 32) if a % 32 < 16 else 2 * ((a % 32) - 16) + 1)
        for a in range(64)]
_TPERM = np.array(_T64 + [t + 64 for t in _T64], dtype=np.int32)


def _sc_aggregate_body(table_h, src_h, dst_h, acc_out, deg_out,
                       src_v, dst_v, bf0, bf1, rows_f, zbuf, ones_v, zdeg,
                       acc_sh, deg_sh, sem0, sem1, osem):
  c = lax.axis_index("c")
  s = lax.axis_index("s")
  base = s * RPT
  zero16 = jnp.zeros((L,), jnp.float32)

  def zrow(r, _):
    for jj in range(DH // L):
      zbuf[r, pl.ds(jj * L, L)] = zero16
    ones_v[r, :] = jnp.ones((L,), jnp.float32)
    return 0
  lax.fori_loop(0, 128, zrow, 0)

  def zdrow(r, _):
    zdeg[r, :] = zero16
    return 0
  lax.fori_loop(0, RPT, zdrow, 0)

  pltpu.sync_copy(zdeg, deg_sh.at[pl.ds(base, RPT)])
  for kk in range(RPT // 128):
    pltpu.sync_copy(zbuf, acc_sh.at[pl.ds(base + kk * 128, 128)])

  pltpu.sync_copy(src_h.at[c, s], src_v)
  pltpu.sync_copy(dst_h.at[s], dst_v)
  plsc.subcore_barrier()

  def issue(j, buf, sem):
    pltpu.async_copy(table_h.at[src_v.at[j]], buf, sem)

  def wait(buf, sem):
    pltpu.make_async_copy(table_h.at[src_v.at[0]], buf, sem).wait()

  def convert(buf):
    def crow(r, _):
      for half in range(2):
        r2 = r * 2 + half
        for g in range(DH // 32):
          v = buf[r2, pl.ds(g * 32, 32)]
          a, b = plsc.unpack(v, format=plsc.PackFormat.INTERLEAVED)
          rows_f[r2, pl.ds(g * 32, L)] = a
          rows_f[r2, pl.ds(g * 32 + L, L)] = b
      return 0
    lax.fori_loop(0, K // 2, crow, 0)

  def scatter(j):
    pltpu.async_copy(ones_v, deg_sh.at[dst_v.at[j]], osem, add=True)
    pltpu.sync_copy(rows_f, acc_sh.at[dst_v.at[j]], add=True)

  issue(0, bf0, sem0)
  issue(1, bf1, sem1)

  def pair(i, _):
    j = i * 2
    wait(bf0, sem0)
    convert(bf0)
    issue(j + 2, bf0, sem0)
    scatter(j)
    wait(bf1, sem1)
    convert(bf1)
    issue(j + 3, bf1, sem1)
    scatter(j + 1)
    return 0
  lax.fori_loop(0, (CH - 2) // 2, pair, 0)
  wait(bf0, sem0)
  convert(bf0)
  scatter(CH - 2)
  wait(bf1, sem1)
  convert(bf1)
  scatter(CH - 1)

  def drain(i, _):
    pltpu.make_async_copy(ones_v, deg_sh.at[dst_v.at[0]], osem).wait()
    return 0
  lax.fori_loop(0, CH, drain, 0)

  plsc.subcore_barrier()
  pltpu.sync_copy(acc_sh.at[pl.ds(base, RPT)], acc_out.at[c, pl.ds(base, RPT)])
  pltpu.sync_copy(deg_sh.at[pl.ds(base, RPT)], deg_out.at[c, pl.ds(base, RPT)])


_sc_aggregate = pl.kernel(
    _sc_aggregate_body,
    out_type=[
        jax.ShapeDtypeStruct((NC, N_ACC, DH), jnp.float32),
        jax.ShapeDtypeStruct((NC, N_ACC, L), jnp.float32),
    ],
    mesh=plsc.VectorSubcoreMesh(
        core_axis_name="c", subcore_axis_name="s", num_cores=NC, num_subcores=NS
    ),
    scratch_types=[
        pltpu.VMEM((CH, K), jnp.int32),
        pltpu.VMEM((CH, K), jnp.int32),
        pltpu.VMEM((K, DH), jnp.bfloat16),
        pltpu.VMEM((K, DH), jnp.bfloat16),
        pltpu.VMEM((K, DH), jnp.float32),
        pltpu.VMEM((128, DH), jnp.float32),
        pltpu.VMEM((K, L), jnp.float32),
        pltpu.VMEM((RPT, L), jnp.float32),
        pltpu.VMEM_SHARED((N_ACC, DH), jnp.float32),
        pltpu.VMEM_SHARED((N_ACC, L), jnp.float32),
        pltpu.SemaphoreType.DMA,
        pltpu.SemaphoreType.DMA,
        pltpu.SemaphoreType.DMA,
    ],
    compiler_params=pltpu.CompilerParams(use_tc_tiling_on_sc=False, needs_layout_passes=False),
)


BR = 2000
NG = N // BR


def _tc_layer1_body(x_b, acc_b, deg_b, ws_b, wn_b, b_b, out_b):
  xv = x_b[...]
  accv = acc_b[...]
  x = jnp.concatenate([xv[0], xv[1]], axis=1)
  acc = jnp.concatenate([accv[0], accv[1]], axis=1)
  deg = jnp.maximum(deg_b[0, :, 0:1], 1.0)
  mean = acc / deg
  h = x @ ws_b[...] + mean @ wn_b[...] + b_b[...]
  h = jnp.maximum(h, 0.0)
  out_b[...] = jnp.stack([h[:, :DH], h[:, DH:]]).astype(jnp.bfloat16)


def _tc_layer1(xs, acc, deg, w_self, w_neigh, b):
  return pl.pallas_call(
      _tc_layer1_body,
      grid=(NG,),
      in_specs=[
          pl.BlockSpec((NC, BR, DH), lambda i: (0, i, 0)),
          pl.BlockSpec((NC, BR, DH), lambda i: (0, i, 0)),
          pl.BlockSpec((1, BR, L), lambda i: (0, i, 0)),
          pl.BlockSpec((D, H), lambda i: (0, 0)),
          pl.BlockSpec((D, H), lambda i: (0, 0)),
          pl.BlockSpec((1, H), lambda i: (0, 0)),
      ],
      out_specs=pl.BlockSpec((NC, BR, DH), lambda i: (0, i, 0)),
      out_shape=jax.ShapeDtypeStruct((NC, N, DH), jnp.bfloat16),
  )(xs, acc, deg, w_self, w_neigh, b)


def _tc_layer2_body(h_b, acc_b, deg_b, ws_b, wn_b, b_b, wc_b, bc_b, out_b, sum_ref):
  i = pl.program_id(0)
  hv = h_b[...]
  accv = acc_b[...]
  h1 = jnp.concatenate([hv[0], hv[1]], axis=1).astype(jnp.float32)
  acc = jnp.concatenate([accv[0], accv[1]], axis=1)
  deg = jnp.maximum(deg_b[0, :, 0:1], 1.0)
  mean = acc / deg
  h = h1 @ ws_b[...] + mean @ wn_b[...] + b_b[...]
  h = jnp.maximum(h, 0.0)

  @pl.when(i == 0)
  def _():
    sum_ref[...] = jnp.zeros_like(sum_ref)

  sum_ref[...] += jnp.sum(h, axis=0, keepdims=True)

  @pl.when(i == NG - 1)
  def _():
    hg = sum_ref[...] * (1.0 / N)
    out_b[...] = hg @ wc_b[...] + bc_b[...]


def _tc_layer2(h1s, acc, deg, w_self, w_neigh, b, w_cls, b_cls):
  return pl.pallas_call(
      _tc_layer2_body,
      grid=(NG,),
      in_specs=[
          pl.BlockSpec((NC, BR, DH), lambda i: (0, i, 0)),
          pl.BlockSpec((NC, BR, DH), lambda i: (0, i, 0)),
          pl.BlockSpec((1, BR, L), lambda i: (0, i, 0)),
          pl.BlockSpec((H, H), lambda i: (0, 0)),
          pl.BlockSpec((H, H), lambda i: (0, 0)),
          pl.BlockSpec((1, H), lambda i: (0, 0)),
          pl.BlockSpec((H, C), lambda i: (0, 0)),
          pl.BlockSpec((1, C), lambda i: (0, 0)),
      ],
      out_specs=pl.BlockSpec((1, C), lambda i: (0, 0)),
      out_shape=jax.ShapeDtypeStruct((1, C), jnp.float32),
      scratch_shapes=[pltpu.VMEM((1, H), jnp.float32)],
  )(h1s, acc, deg, w_self, w_neigh, b, w_cls, b_cls)


@jax.jit
def kernel(x, edge_index, W_self1, W_neigh1, b1, W_self2, W_neigh2, b2, W_cls, b_cls):
  src = edge_index[0]
  dst = edge_index[1]
  pad = E_PAD - E
  src_p = jnp.concatenate([src, jnp.zeros((pad,), jnp.int32)])
  dst_p = jnp.concatenate([dst, jnp.full((pad,), N, jnp.int32)])
  src4 = (src_p[None, :] + (jnp.arange(NC, dtype=jnp.int32) * N)[:, None])
  src4 = src4.reshape(NC, NS, CH, K)
  dst3 = dst_p.reshape(NS, CH, K)

  xs = jnp.stack([x[:, :DH], x[:, DH:]])
  xsb = xs.astype(jnp.bfloat16)
  perm = jnp.asarray(_TPERM)
  acc1, deg = _sc_aggregate(xsb.reshape(NC * N, DH), src4, dst3)
  h1s = _tc_layer1(xs, acc1, deg, W_self1, W_neigh1[perm], b1.reshape(1, H))
  acc2, _ = _sc_aggregate(h1s.reshape(NC * N, DH), src4, dst3)
  return _tc_layer2(h1s, acc2, deg, W_self2, W_neigh2[perm], b2.reshape(1, H),
                    W_cls, b_cls.reshape(1, C))


# speedup vs baseline: 1.3623x; 1.0092x over previous
import functools

import numpy as np
import jax
import jax.numpy as jnp
from jax import lax
from jax.experimental import pallas as pl
from jax.experimental.pallas import tpu as pltpu
from jax.experimental.pallas import tpu_sc as plsc

N = 10000
D = 128
H = 128
C = 10
E = 320000

NC = 2
NS = 16
L = 16
DGW = 8
DH = D // NC

K = 128
E_PAD = 327680
CH = E_PAD // (NS * K)
N_ACC = 10240
RPT = N_ACC // NS

_T64 = [(a // 32) * 32 + (2 * (a % 32) if a % 32 < 16 else 2 * ((a % 32) - 16) + 1)
        for a in range(64)]
_TPERM = np.array(_T64 + [t + 64 for t in _T64], dtype=np.int32)


def _sc_aggregate_body(table_h, src_h, dst_h, ones_h, zdeg_h, acc_out, deg_out,
                       src_v, dst_v, bf0, bf1, f0, f1, zbuf, ones_v,
                       acc_sh, deg_sh, sem0, sem1, osem):
  c = lax.axis_index("c")
  s = lax.axis_index("s")
  base = s * RPT
  zero16 = jnp.zeros((L,), jnp.float32)

  def zrow(r, _):
    for jj in range(DH // L):
      zbuf[r, pl.ds(jj * L, L)] = zero16
    return 0
  lax.fori_loop(0, 128, zrow, 0)

  pltpu.sync_copy(ones_h, ones_v)
  pltpu.sync_copy(zdeg_h, deg_sh.at[pl.ds(base, RPT)])
  for kk in range(RPT // 128):
    pltpu.sync_copy(zbuf, acc_sh.at[pl.ds(base + kk * 128, 128)])

  pltpu.sync_copy(src_h.at[c, s], src_v)
  pltpu.sync_copy(dst_h.at[s], dst_v)
  plsc.subcore_barrier()

  def issue(j, buf, sem):
    pltpu.async_copy(table_h.at[src_v.at[j]], buf, sem)

  def wait(buf, sem):
    pltpu.make_async_copy(table_h.at[src_v.at[0]], buf, sem).wait()

  def convert(buf, fbuf):
    def crow(r, _):
      for half in range(2):
        r2 = r * 2 + half
        for g in range(DH // 32):
          v = buf[r2, pl.ds(g * 32, 32)]
          a, b = plsc.unpack(v, format=plsc.PackFormat.INTERLEAVED)
          fbuf[r2, pl.ds(g * 32, L)] = a
          fbuf[r2, pl.ds(g * 32 + L, L)] = b
      return 0
    lax.fori_loop(0, K // 2, crow, 0)

  def scatter(j, fbuf):
    pltpu.async_copy(ones_v, deg_sh.at[dst_v.at[j]], osem, add=True)
    pltpu.sync_copy(fbuf, acc_sh.at[dst_v.at[j]], add=True)

  issue(0, bf0, sem0)
  issue(1, bf1, sem1)

  def pair(i, _):
    j = i * 2
    wait(bf0, sem0)
    convert(bf0, f0)

    @pl.when(j + 2 < CH)
    def _():
      issue(j + 2, bf0, sem0)

    scatter(j, f0)
    wait(bf1, sem1)
    convert(bf1, f1)

    @pl.when(j + 3 < CH)
    def _():
      issue(j + 3, bf1, sem1)

    scatter(j + 1, f1)
    return 0
  lax.fori_loop(0, CH // 2, pair, 0)

  def drain(i, _):
    pltpu.make_async_copy(ones_v, deg_sh.at[dst_v.at[0]], osem).wait()
    return 0
  lax.fori_loop(0, CH, drain, 0)

  plsc.subcore_barrier()
  pltpu.sync_copy(acc_sh.at[pl.ds(base, RPT)], acc_out.at[c, pl.ds(base, RPT)])
  pltpu.sync_copy(deg_sh.at[pl.ds(base, RPT)], deg_out.at[c, pl.ds(base, RPT)])


_sc_aggregate = pl.kernel(
    _sc_aggregate_body,
    out_type=[
        jax.ShapeDtypeStruct((NC, N_ACC, DH), jnp.float32),
        jax.ShapeDtypeStruct((NC, N_ACC, DGW), jnp.float32),
    ],
    mesh=plsc.VectorSubcoreMesh(
        core_axis_name="c", subcore_axis_name="s", num_cores=NC, num_subcores=NS
    ),
    scratch_types=[
        pltpu.VMEM((CH, K), jnp.int32),
        pltpu.VMEM((CH, K), jnp.int32),
        pltpu.VMEM((K, DH), jnp.bfloat16),
        pltpu.VMEM((K, DH), jnp.bfloat16),
        pltpu.VMEM((K, DH), jnp.float32),
        pltpu.VMEM((K, DH), jnp.float32),
        pltpu.VMEM((128, DH), jnp.float32),
        pltpu.VMEM((K, DGW), jnp.float32),
        pltpu.VMEM_SHARED((N_ACC, DH), jnp.float32),
        pltpu.VMEM_SHARED((N_ACC, DGW), jnp.float32),
        pltpu.SemaphoreType.DMA,
        pltpu.SemaphoreType.DMA,
        pltpu.SemaphoreType.DMA,
    ],
    compiler_params=pltpu.CompilerParams(use_tc_tiling_on_sc=False, needs_layout_passes=False),
)


BR = 2000
NG = N // BR


def _tc_layer1_body(x_b, acc_b, deg_b, ws_b, wn_b, b_b, out_b):
  xv = x_b[...]
  accv = acc_b[...]
  x = jnp.concatenate([xv[0], xv[1]], axis=1)
  acc = jnp.concatenate([accv[0], accv[1]], axis=1)
  deg = jnp.maximum(deg_b[0, :, 0:1], 1.0)
  mean = acc / deg
  h = x @ ws_b[...] + mean @ wn_b[...] + b_b[...]
  h = jnp.maximum(h, 0.0)
  out_b[...] = jnp.stack([h[:, :DH], h[:, DH:]]).astype(jnp.bfloat16)


def _tc_layer1(xs, acc, deg, w_self, w_neigh, b):
  return pl.pallas_call(
      _tc_layer1_body,
      grid=(NG,),
      in_specs=[
          pl.BlockSpec((NC, BR, DH), lambda i: (0, i, 0)),
          pl.BlockSpec((NC, BR, DH), lambda i: (0, i, 0)),
          pl.BlockSpec((1, BR, DGW), lambda i: (0, i, 0)),
          pl.BlockSpec((D, H), lambda i: (0, 0)),
          pl.BlockSpec((D, H), lambda i: (0, 0)),
          pl.BlockSpec((1, H), lambda i: (0, 0)),
      ],
      out_specs=pl.BlockSpec((NC, BR, DH), lambda i: (0, i, 0)),
      out_shape=jax.ShapeDtypeStruct((NC, N, DH), jnp.bfloat16),
  )(xs, acc, deg, w_self, w_neigh, b)


def _tc_layer2_body(h_b, acc_b, deg_b, ws_b, wn_b, b_b, wc_b, bc_b, out_b, sum_ref):
  i = pl.program_id(0)
  hv = h_b[...]
  accv = acc_b[...]
  h1 = jnp.concatenate([hv[0], hv[1]], axis=1).astype(jnp.float32)
  acc = jnp.concatenate([accv[0], accv[1]], axis=1)
  deg = jnp.maximum(deg_b[0, :, 0:1], 1.0)
  mean = acc / deg
  h = h1 @ ws_b[...] + mean @ wn_b[...] + b_b[...]
  h = jnp.maximum(h, 0.0)

  @pl.when(i == 0)
  def _():
    sum_ref[...] = jnp.zeros_like(sum_ref)

  sum_ref[...] += jnp.sum(h, axis=0, keepdims=True)

  @pl.when(i == NG - 1)
  def _():
    hg = sum_ref[...] * (1.0 / N)
    out_b[...] = hg @ wc_b[...] + bc_b[...]


def _tc_layer2(h1s, acc, deg, w_self, w_neigh, b, w_cls, b_cls):
  return pl.pallas_call(
      _tc_layer2_body,
      grid=(NG,),
      in_specs=[
          pl.BlockSpec((NC, BR, DH), lambda i: (0, i, 0)),
          pl.BlockSpec((NC, BR, DH), lambda i: (0, i, 0)),
          pl.BlockSpec((1, BR, DGW), lambda i: (0, i, 0)),
          pl.BlockSpec((H, H), lambda i: (0, 0)),
          pl.BlockSpec((H, H), lambda i: (0, 0)),
          pl.BlockSpec((1, H), lambda i: (0, 0)),
          pl.BlockSpec((H, C), lambda i: (0, 0)),
          pl.BlockSpec((1, C), lambda i: (0, 0)),
      ],
      out_specs=pl.BlockSpec((1, C), lambda i: (0, 0)),
      out_shape=jax.ShapeDtypeStruct((1, C), jnp.float32),
      scratch_shapes=[pltpu.VMEM((1, H), jnp.float32)],
  )(h1s, acc, deg, w_self, w_neigh, b, w_cls, b_cls)


@jax.jit
def kernel(x, edge_index, W_self1, W_neigh1, b1, W_self2, W_neigh2, b2, W_cls, b_cls):
  src = edge_index[0]
  dst = edge_index[1]
  pad = E_PAD - E
  src_p = jnp.concatenate([src, jnp.zeros((pad,), jnp.int32)])
  dst_p = jnp.concatenate([dst, jnp.full((pad,), N, jnp.int32)])
  src4 = (src_p[None, :] + (jnp.arange(NC, dtype=jnp.int32) * N)[:, None])
  src4 = src4.reshape(NC, NS, CH, K)
  dst3 = dst_p.reshape(NS, CH, K)

  xs = jnp.stack([x[:, :DH], x[:, DH:]])
  xsb = xs.astype(jnp.bfloat16)
  perm = jnp.asarray(_TPERM)
  ones_h = jnp.ones((K, DGW), jnp.float32)
  zdeg_h = jnp.zeros((RPT, DGW), jnp.float32)
  acc1, deg = _sc_aggregate(xsb.reshape(NC * N, DH), src4, dst3, ones_h, zdeg_h)
  h1s = _tc_layer1(xs, acc1, deg, W_self1, W_neigh1[perm], b1.reshape(1, H))
  acc2, _ = _sc_aggregate(h1s.reshape(NC * N, DH), src4, dst3, ones_h, zdeg_h)
  return _tc_layer2(h1s, acc2, deg, W_self2, W_neigh2[perm], b2.reshape(1, H),
                    W_cls, b_cls.reshape(1, C))


# speedup vs baseline: 1.5251x; 1.1195x over previous
import functools

import numpy as np
import jax
import jax.numpy as jnp
from jax import lax
from jax.experimental import pallas as pl
from jax.experimental.pallas import tpu as pltpu
from jax.experimental.pallas import tpu_sc as plsc

N = 10000
D = 128
H = 128
C = 10
E = 320000

NC = 2
NS = 16
L = 16
DGW = 8
DH = D // NC

K = 128
E_PAD = 327680
CH = E_PAD // (NS * K)
N_ACC = 10240
RPT = N_ACC // NS

_T64 = [(a // 32) * 32 + (2 * (a % 32) if a % 32 < 16 else 2 * ((a % 32) - 16) + 1)
        for a in range(64)]
_TPERM = np.array(_T64 + [t + 64 for t in _T64], dtype=np.int32)


def _sc_aggregate_body(table_h, src_h, dst_h, ones_h, zdeg_h, acc_out, deg_out,
                       src_v, dst_v, bf0, bf1, f0, f1, zbuf, ones_v,
                       acc_sh, deg_sh, sem0, sem1, ss0, ss1, osem):
  c = lax.axis_index("c")
  s = lax.axis_index("s")
  base = s * RPT
  zero16 = jnp.zeros((L,), jnp.float32)

  def zrow(r, _):
    for jj in range(DH // L):
      zbuf[r, pl.ds(jj * L, L)] = zero16
    return 0
  lax.fori_loop(0, 128, zrow, 0)

  pltpu.sync_copy(ones_h, ones_v)
  pltpu.sync_copy(zdeg_h, deg_sh.at[pl.ds(base, RPT)])
  for kk in range(RPT // 128):
    pltpu.sync_copy(zbuf, acc_sh.at[pl.ds(base + kk * 128, 128)])

  pltpu.sync_copy(src_h.at[c, s], src_v)
  pltpu.sync_copy(dst_h.at[s], dst_v)
  plsc.subcore_barrier()

  def issue(j, buf, sem):
    pltpu.async_copy(table_h.at[src_v.at[j]], buf, sem)

  def wait(buf, sem):
    pltpu.make_async_copy(table_h.at[src_v.at[0]], buf, sem).wait()

  def convert(buf, fbuf):
    def crow(r, _):
      for half in range(2):
        r2 = r * 2 + half
        for g in range(DH // 32):
          v = buf[r2, pl.ds(g * 32, 32)]
          a, b = plsc.unpack(v, format=plsc.PackFormat.INTERLEAVED)
          fbuf[r2, pl.ds(g * 32, L)] = a
          fbuf[r2, pl.ds(g * 32 + L, L)] = b
      return 0
    lax.fori_loop(0, K // 2, crow, 0)

  def scatter(j, fbuf, ssem):
    pltpu.async_copy(ones_v, deg_sh.at[dst_v.at[j]], osem, add=True)
    pltpu.async_copy(fbuf, acc_sh.at[dst_v.at[j]], ssem, add=True)

  def wait_scatter(fbuf, ssem):
    pltpu.make_async_copy(fbuf, acc_sh.at[dst_v.at[0]], ssem).wait()

  issue(0, bf0, sem0)
  issue(1, bf1, sem1)

  def pair(i, _):
    j = i * 2
    wait(bf0, sem0)

    @pl.when(i > 0)
    def _():
      wait_scatter(f0, ss0)

    convert(bf0, f0)

    @pl.when(j + 2 < CH)
    def _():
      issue(j + 2, bf0, sem0)

    scatter(j, f0, ss0)
    wait(bf1, sem1)

    @pl.when(i > 0)
    def _():
      wait_scatter(f1, ss1)

    convert(bf1, f1)

    @pl.when(j + 3 < CH)
    def _():
      issue(j + 3, bf1, sem1)

    scatter(j + 1, f1, ss1)
    return 0
  lax.fori_loop(0, CH // 2, pair, 0)
  wait_scatter(f0, ss0)
  wait_scatter(f1, ss1)

  def drain(i, _):
    pltpu.make_async_copy(ones_v, deg_sh.at[dst_v.at[0]], osem).wait()
    return 0
  lax.fori_loop(0, CH, drain, 0)

  plsc.subcore_barrier()
  pltpu.sync_copy(acc_sh.at[pl.ds(base, RPT)], acc_out.at[c, pl.ds(base, RPT)])
  pltpu.sync_copy(deg_sh.at[pl.ds(base, RPT)], deg_out.at[c, pl.ds(base, RPT)])


_sc_aggregate = pl.kernel(
    _sc_aggregate_body,
    out_type=[
        jax.ShapeDtypeStruct((NC, N_ACC, DH), jnp.float32),
        jax.ShapeDtypeStruct((NC, N_ACC, DGW), jnp.float32),
    ],
    mesh=plsc.VectorSubcoreMesh(
        core_axis_name="c", subcore_axis_name="s", num_cores=NC, num_subcores=NS
    ),
    scratch_types=[
        pltpu.VMEM((CH, K), jnp.int32),
        pltpu.VMEM((CH, K), jnp.int32),
        pltpu.VMEM((K, DH), jnp.bfloat16),
        pltpu.VMEM((K, DH), jnp.bfloat16),
        pltpu.VMEM((K, DH), jnp.float32),
        pltpu.VMEM((K, DH), jnp.float32),
        pltpu.VMEM((128, DH), jnp.float32),
        pltpu.VMEM((K, DGW), jnp.float32),
        pltpu.VMEM_SHARED((N_ACC, DH), jnp.float32),
        pltpu.VMEM_SHARED((N_ACC, DGW), jnp.float32),
        pltpu.SemaphoreType.DMA,
        pltpu.SemaphoreType.DMA,
        pltpu.SemaphoreType.DMA,
        pltpu.SemaphoreType.DMA,
        pltpu.SemaphoreType.DMA,
    ],
    compiler_params=pltpu.CompilerParams(use_tc_tiling_on_sc=False, needs_layout_passes=False),
)


BR = 2000
NG = N // BR


def _tc_layer1_body(x_b, acc_b, deg_b, ws_b, wn_b, b_b, out_b):
  xv = x_b[...]
  accv = acc_b[...]
  x = jnp.concatenate([xv[0], xv[1]], axis=1)
  acc = jnp.concatenate([accv[0], accv[1]], axis=1)
  deg = jnp.maximum(deg_b[0, :, 0:1], 1.0)
  mean = acc / deg
  h = x @ ws_b[...] + mean @ wn_b[...] + b_b[...]
  h = jnp.maximum(h, 0.0)
  out_b[...] = jnp.stack([h[:, :DH], h[:, DH:]]).astype(jnp.bfloat16)


def _tc_layer1(xs, acc, deg, w_self, w_neigh, b):
  return pl.pallas_call(
      _tc_layer1_body,
      grid=(NG,),
      in_specs=[
          pl.BlockSpec((NC, BR, DH), lambda i: (0, i, 0)),
          pl.BlockSpec((NC, BR, DH), lambda i: (0, i, 0)),
          pl.BlockSpec((1, BR, DGW), lambda i: (0, i, 0)),
          pl.BlockSpec((D, H), lambda i: (0, 0)),
          pl.BlockSpec((D, H), lambda i: (0, 0)),
          pl.BlockSpec((1, H), lambda i: (0, 0)),
      ],
      out_specs=pl.BlockSpec((NC, BR, DH), lambda i: (0, i, 0)),
      out_shape=jax.ShapeDtypeStruct((NC, N, DH), jnp.bfloat16),
  )(xs, acc, deg, w_self, w_neigh, b)


def _tc_layer2_body(h_b, acc_b, deg_b, ws_b, wn_b, b_b, wc_b, bc_b, out_b, sum_ref):
  i = pl.program_id(0)
  hv = h_b[...]
  accv = acc_b[...]
  h1 = jnp.concatenate([hv[0], hv[1]], axis=1).astype(jnp.float32)
  acc = jnp.concatenate([accv[0], accv[1]], axis=1)
  deg = jnp.maximum(deg_b[0, :, 0:1], 1.0)
  mean = acc / deg
  h = h1 @ ws_b[...] + mean @ wn_b[...] + b_b[...]
  h = jnp.maximum(h, 0.0)

  @pl.when(i == 0)
  def _():
    sum_ref[...] = jnp.zeros_like(sum_ref)

  sum_ref[...] += jnp.sum(h, axis=0, keepdims=True)

  @pl.when(i == NG - 1)
  def _():
    hg = sum_ref[...] * (1.0 / N)
    out_b[...] = hg @ wc_b[...] + bc_b[...]


def _tc_layer2(h1s, acc, deg, w_self, w_neigh, b, w_cls, b_cls):
  return pl.pallas_call(
      _tc_layer2_body,
      grid=(NG,),
      in_specs=[
          pl.BlockSpec((NC, BR, DH), lambda i: (0, i, 0)),
          pl.BlockSpec((NC, BR, DH), lambda i: (0, i, 0)),
          pl.BlockSpec((1, BR, DGW), lambda i: (0, i, 0)),
          pl.BlockSpec((H, H), lambda i: (0, 0)),
          pl.BlockSpec((H, H), lambda i: (0, 0)),
          pl.BlockSpec((1, H), lambda i: (0, 0)),
          pl.BlockSpec((H, C), lambda i: (0, 0)),
          pl.BlockSpec((1, C), lambda i: (0, 0)),
      ],
      out_specs=pl.BlockSpec((1, C), lambda i: (0, 0)),
      out_shape=jax.ShapeDtypeStruct((1, C), jnp.float32),
      scratch_shapes=[pltpu.VMEM((1, H), jnp.float32)],
  )(h1s, acc, deg, w_self, w_neigh, b, w_cls, b_cls)


@jax.jit
def kernel(x, edge_index, W_self1, W_neigh1, b1, W_self2, W_neigh2, b2, W_cls, b_cls):
  src = edge_index[0]
  dst = edge_index[1]
  pad = E_PAD - E
  src_p = jnp.concatenate([src, jnp.zeros((pad,), jnp.int32)])
  dst_p = jnp.concatenate([dst, jnp.full((pad,), N, jnp.int32)])
  src4 = (src_p[None, :] + (jnp.arange(NC, dtype=jnp.int32) * N)[:, None])
  src4 = src4.reshape(NC, NS, CH, K)
  dst3 = dst_p.reshape(NS, CH, K)

  xs = jnp.stack([x[:, :DH], x[:, DH:]])
  xsb = xs.astype(jnp.bfloat16)
  perm = jnp.asarray(_TPERM)
  ones_h = jnp.ones((K, DGW), jnp.float32)
  zdeg_h = jnp.zeros((RPT, DGW), jnp.float32)
  acc1, deg = _sc_aggregate(xsb.reshape(NC * N, DH), src4, dst3, ones_h, zdeg_h)
  h1s = _tc_layer1(xs, acc1, deg, W_self1, W_neigh1[perm], b1.reshape(1, H))
  acc2, _ = _sc_aggregate(h1s.reshape(NC * N, DH), src4, dst3, ones_h, zdeg_h)
  return _tc_layer2(h1s, acc2, deg, W_self2, W_neigh2[perm], b2.reshape(1, H),
                    W_cls, b_cls.reshape(1, C))
